# baseline (speedup 1.0000x reference)
import functools

import jax
import jax.numpy as jnp
from jax import lax
from jax.experimental import pallas as pl
from jax.experimental.pallas import tpu as pltpu
from jax.experimental.pallas import tpu_sc as plsc

HIDDEN = 64
L = 16
NC, NS = 2, 16
NW = NC * NS
CHUNK = 128
BCHUNK = 128
RBLK = 16384


def _bias_row_sums(ub, ib):
    n = ub.shape[0]

    def body(ub_ref, ib_ref, ubs_ref, ibs_ref):
        ubs_ref[...] = jnp.sum(ub_ref[...], axis=1)
        ibs_ref[...] = jnp.sum(ib_ref[...], axis=1)

    return pl.pallas_call(
        body,
        grid=(pl.cdiv(n, RBLK),),
        in_specs=[
            pl.BlockSpec((RBLK, HIDDEN), lambda i: (i, 0)),
            pl.BlockSpec((RBLK, HIDDEN), lambda i: (i, 0)),
        ],
        out_specs=[
            pl.BlockSpec((RBLK,), lambda i: (i,)),
            pl.BlockSpec((RBLK,), lambda i: (i,)),
        ],
        out_shape=[
            jax.ShapeDtypeStruct((n,), jnp.float32),
            jax.ShapeDtypeStruct((n,), jnp.float32),
        ],
    )(ub, ib)


@functools.partial(jax.jit, static_argnames=("B",))
def _run(user, item, user_factors, item_factors, user_biases, item_biases, B):
    b_per_w = B // NW
    n_chunks = b_per_w // CHUNK
    mesh = plsc.VectorSubcoreMesh(core_axis_name="c", subcore_axis_name="s")

    ubs, ibs = _bias_row_sums(user_biases, item_biases)

    @functools.partial(
        pl.kernel,
        mesh=mesh,
        compiler_params=pltpu.CompilerParams(
            needs_layout_passes=False, use_tc_tiling_on_sc=False,
            skip_device_barrier=True, disable_bounds_checks=True,
            disable_semaphore_checks=True),
        out_type=jax.ShapeDtypeStruct((B,), jnp.float32),
        scratch_types=[
            pltpu.VMEM((CHUNK,), jnp.int32),
            pltpu.VMEM((CHUNK,), jnp.int32),
            pltpu.VMEM((CHUNK, HIDDEN), jnp.float32),
            pltpu.VMEM((CHUNK, HIDDEN), jnp.float32),
            pltpu.VMEM((CHUNK,), jnp.float32),
            pltpu.SemaphoreType.DMA,
        ],
    )
    def kdot(user_hbm, item_hbm, uf_hbm, if_hbm, out_hbm,
             uidx_v, iidx_v, uf_v, if_v, o_v, sem):
        wid = lax.axis_index("s") * NC + lax.axis_index("c")
        base = wid * b_per_w
        lane = lax.iota(jnp.int32, L)

        def chunk_body(c, _):
            off = base + c * CHUNK
            pltpu.sync_copy(user_hbm.at[pl.ds(off, CHUNK)], uidx_v)
            pltpu.sync_copy(item_hbm.at[pl.ds(off, CHUNK)], iidx_v)
            cu = pltpu.async_copy(uf_hbm.at[uidx_v], uf_v, sem)
            ci = pltpu.async_copy(if_hbm.at[iidx_v], if_v, sem)
            cu.wait()
            ci.wait()

            for g in range(CHUNK // L):
                vec = jnp.zeros((L,), jnp.float32)
                for jj in range(L):
                    j = g * L + jj
                    acc = jnp.zeros((L,), jnp.float32)
                    for kk in range(HIDDEN // L):
                        sl = pl.ds(kk * L, L)
                        acc = acc + uf_v[j, sl] * if_v[j, sl]
                    vec = jnp.where(lane == jj, jnp.sum(acc), vec)
                o_v[pl.ds(g * L, L)] = vec
            pltpu.sync_copy(o_v, out_hbm.at[pl.ds(off, CHUNK)])
            return 0

        lax.fori_loop(0, n_chunks, chunk_body, 0)

    dotp = kdot(user, item, user_factors, item_factors)

    n_bchunks = b_per_w // BCHUNK

    @functools.partial(
        pl.kernel,
        mesh=mesh,
        compiler_params=pltpu.CompilerParams(
            needs_layout_passes=False, use_tc_tiling_on_sc=False,
            skip_device_barrier=True, disable_bounds_checks=True,
            disable_semaphore_checks=True),
        out_type=jax.ShapeDtypeStruct((B,), jnp.float32),
        scratch_types=[
            pltpu.VMEM((BCHUNK,), jnp.int32),
            pltpu.VMEM((BCHUNK,), jnp.int32),
            pltpu.VMEM((BCHUNK,), jnp.float32),
            pltpu.VMEM((BCHUNK,), jnp.float32),
            pltpu.VMEM((BCHUNK,), jnp.float32),
            pltpu.SemaphoreType.DMA,
        ],
    )
    def kbias(user_hbm, item_hbm, ubs_hbm, ibs_hbm, dot_hbm, out_hbm,
              uidx_v, iidx_v, ug_v, ig_v, d_v, sem):
        wid = lax.axis_index("s") * NC + lax.axis_index("c")
        base = wid * b_per_w

        def chunk_body(c, _):
            off = base + c * BCHUNK
            pltpu.sync_copy(user_hbm.at[pl.ds(off, BCHUNK)], uidx_v)
            pltpu.sync_copy(item_hbm.at[pl.ds(off, BCHUNK)], iidx_v)
            cu = pltpu.async_copy(ubs_hbm.at[uidx_v], ug_v, sem)
            ci = pltpu.async_copy(ibs_hbm.at[iidx_v], ig_v, sem)
            pltpu.sync_copy(dot_hbm.at[pl.ds(off, BCHUNK)], d_v)
            cu.wait()
            ci.wait()
            for g in range(BCHUNK // L):
                sl = pl.ds(g * L, L)
                d_v[sl] = d_v[sl] + ug_v[sl] + ig_v[sl]
            pltpu.sync_copy(d_v, out_hbm.at[pl.ds(off, BCHUNK)])
            return 0

        lax.fori_loop(0, n_bchunks, chunk_body, 0)

    return kbias(user, item, ubs, ibs, dotp)


def kernel(user, item, user_factors, item_factors, user_biases, item_biases):
    B = user.shape[0]
    out = _run(user.astype(jnp.int32), item.astype(jnp.int32),
               user_factors, item_factors, user_biases, item_biases, B)
    return out.reshape(B, 1)


# speedup vs baseline: 1.0210x; 1.0210x over previous
import functools

import jax
import jax.numpy as jnp
from jax import lax
from jax.experimental import pallas as pl
from jax.experimental.pallas import tpu as pltpu
from jax.experimental.pallas import tpu_sc as plsc

HIDDEN = 64
L = 16
NC, NS = 2, 16
NW = NC * NS
CHUNK = 128
BCHUNK = 128
RBLK = 16384


def _bias_row_sums(ub, ib):
    n = ub.shape[0]

    def body(ub_ref, ib_ref, ubs_ref, ibs_ref):
        ubs_ref[...] = jnp.sum(ub_ref[...], axis=1)
        ibs_ref[...] = jnp.sum(ib_ref[...], axis=1)

    return pl.pallas_call(
        body,
        grid=(pl.cdiv(n, RBLK),),
        in_specs=[
            pl.BlockSpec((RBLK, HIDDEN), lambda i: (i, 0)),
            pl.BlockSpec((RBLK, HIDDEN), lambda i: (i, 0)),
        ],
        out_specs=[
            pl.BlockSpec((RBLK,), lambda i: (i,)),
            pl.BlockSpec((RBLK,), lambda i: (i,)),
        ],
        out_shape=[
            jax.ShapeDtypeStruct((n,), jnp.float32),
            jax.ShapeDtypeStruct((n,), jnp.float32),
        ],
    )(ub, ib)


@functools.partial(jax.jit, static_argnames=("B",))
def _run(user, item, user_factors, item_factors, user_biases, item_biases, B):
    b_per_w = B // NW
    n_chunks = b_per_w // CHUNK
    mesh = plsc.VectorSubcoreMesh(core_axis_name="c", subcore_axis_name="s")

    ubs, ibs = _bias_row_sums(user_biases, item_biases)

    @functools.partial(
        pl.kernel,
        mesh=mesh,
        compiler_params=pltpu.CompilerParams(
            needs_layout_passes=False, use_tc_tiling_on_sc=False,
            skip_device_barrier=True, disable_bounds_checks=True,
            disable_semaphore_checks=True),
        out_type=jax.ShapeDtypeStruct((B,), jnp.float32),
        scratch_types=[
            pltpu.VMEM((CHUNK,), jnp.int32),
            pltpu.VMEM((CHUNK,), jnp.int32),
            pltpu.VMEM((CHUNK, HIDDEN), jnp.float32),
            pltpu.VMEM((CHUNK, HIDDEN), jnp.float32),
            pltpu.VMEM((CHUNK,), jnp.float32),
            pltpu.VMEM((CHUNK,), jnp.float32),
            pltpu.VMEM((CHUNK,), jnp.float32),
            pltpu.SemaphoreType.DMA,
        ],
    )
    def kdot(user_hbm, item_hbm, uf_hbm, if_hbm, ubs_hbm, ibs_hbm, out_hbm,
             uidx_v, iidx_v, uf_v, if_v, ug_v, ig_v, o_v, sem):
        wid = lax.axis_index("s") * NC + lax.axis_index("c")
        base = wid * b_per_w
        lane = lax.iota(jnp.int32, L)

        def chunk_body(c, _):
            off = base + c * CHUNK
            pltpu.sync_copy(user_hbm.at[pl.ds(off, CHUNK)], uidx_v)
            pltpu.sync_copy(item_hbm.at[pl.ds(off, CHUNK)], iidx_v)
            cu = pltpu.async_copy(uf_hbm.at[uidx_v], uf_v, sem)
            ci = pltpu.async_copy(if_hbm.at[iidx_v], if_v, sem)
            cub = pltpu.async_copy(ubs_hbm.at[uidx_v], ug_v, sem)
            cib = pltpu.async_copy(ibs_hbm.at[iidx_v], ig_v, sem)
            cu.wait()
            ci.wait()
            cub.wait()
            cib.wait()

            for g in range(CHUNK // L):
                vec = jnp.zeros((L,), jnp.float32)
                for jj in range(L):
                    j = g * L + jj
                    acc = jnp.zeros((L,), jnp.float32)
                    for kk in range(HIDDEN // L):
                        sl = pl.ds(kk * L, L)
                        acc = acc + uf_v[j, sl] * if_v[j, sl]
                    vec = jnp.where(lane == jj, jnp.sum(acc), vec)
                gsl = pl.ds(g * L, L)
                o_v[gsl] = vec + ug_v[gsl] + ig_v[gsl]
            pltpu.sync_copy(o_v, out_hbm.at[pl.ds(off, CHUNK)])
            return 0

        lax.fori_loop(0, n_chunks, chunk_body, 0)

    return kdot(user, item, user_factors, item_factors, ubs, ibs)


def kernel(user, item, user_factors, item_factors, user_biases, item_biases):
    B = user.shape[0]
    out = _run(user.astype(jnp.int32), item.astype(jnp.int32),
               user_factors, item_factors, user_biases, item_biases, B)
    return out.reshape(B, 1)
